# baseline (speedup 1.0000x reference)
import functools

import jax
import jax.numpy as jnp
from jax.experimental import pallas as pl
from jax.experimental.pallas import tpu as pltpu

N = 10000
F = 128
H = 64
K = 16


def _proj_kernel(x0, x1, x2, x3, w0, w1, w2, w3, b0, b1, b2, b3, wg0, p_ref):
    xs = (x0, x1, x2, x3)
    ws = (w0, w1, w2, w3)
    bs = (b0, b1, b2, b3)
    g = wg0[...]
    for v in range(4):
        z = jnp.maximum(
            jnp.dot(xs[v][...], ws[v][...], preferred_element_type=jnp.float32)
            + bs[v][...],
            0.0,
        )
        p_ref[:, v * H:(v + 1) * H] = jnp.dot(
            z, g, preferred_element_type=jnp.float32
        ).astype(jnp.bfloat16)


def _project(t0, t1, m0, m1, Wps, bps, Wg0):
    bm = 2000
    grid = (N // bm,)
    row = lambda i: (i, 0)
    full = lambda i: (0, 0)
    return pl.pallas_call(
        _proj_kernel,
        grid=grid,
        in_specs=[pl.BlockSpec((bm, F), row)] * 4
        + [pl.BlockSpec((F, H), full)] * 4
        + [pl.BlockSpec((1, H), full)] * 4
        + [pl.BlockSpec((H, H), full)],
        out_specs=pl.BlockSpec((bm, 4 * H), row),
        out_shape=jax.ShapeDtypeStruct((N, 4 * H), jnp.bfloat16),
        compiler_params=pltpu.CompilerParams(
            dimension_semantics=("parallel",)
        ),
    )(t0, t1, m0, m1, *Wps, *bps, Wg0)


def _agg_kernel_l1(a_ref, x_ref, wg1_ref, o_ref, abf_ref, *, cb):
    a_bf = a_ref[...].astype(jnp.bfloat16)
    abf_ref[...] = a_bf
    acc = jnp.dot(a_bf, x_ref[...], preferred_element_type=jnp.float32)
    g = wg1_ref[...]
    for j in range(cb // H):
        o_ref[:, j * H:(j + 1) * H] = jnp.dot(
            jnp.maximum(acc[:, j * H:(j + 1) * H], 0.0),
            g,
            preferred_element_type=jnp.float32,
        ).astype(jnp.bfloat16)


def _agg_kernel_l2(a_ref, x_ref, o_ref):
    o_ref[...] = jnp.dot(
        a_ref[...], x_ref[...], preferred_element_type=jnp.float32
    )


def _aggregate_l1(A, X, Wg1, col_block=None, col_index=0, bm=400):
    cb = col_block if col_block is not None else X.shape[1]
    return pl.pallas_call(
        functools.partial(_agg_kernel_l1, cb=cb),
        grid=(N // bm,),
        in_specs=[
            pl.BlockSpec((bm, N), lambda i: (i, 0)),
            pl.BlockSpec((N, cb), lambda i: (0, col_index)),
            pl.BlockSpec((H, H), lambda i: (0, 0)),
        ],
        out_specs=[
            pl.BlockSpec((bm, cb), lambda i: (i, 0)),
            pl.BlockSpec((bm, N), lambda i: (i, 0)),
        ],
        out_shape=[
            jax.ShapeDtypeStruct((N, cb), jnp.bfloat16),
            jax.ShapeDtypeStruct((N, N), jnp.bfloat16),
        ],
        compiler_params=pltpu.CompilerParams(
            dimension_semantics=("parallel",),
            vmem_limit_bytes=120 * 1024 * 1024,
        ),
    )(A, X, Wg1)


def _aggregate_l2(A_bf, X, bm=1000):
    cb = X.shape[1]
    return pl.pallas_call(
        _agg_kernel_l2,
        grid=(N // bm,),
        in_specs=[
            pl.BlockSpec((bm, N), lambda i: (i, 0)),
            pl.BlockSpec((N, cb), lambda i: (0, 0)),
        ],
        out_specs=pl.BlockSpec((bm, cb), lambda i: (i, 0)),
        out_shape=jax.ShapeDtypeStruct((N, cb), jnp.float32),
        compiler_params=pltpu.CompilerParams(
            dimension_semantics=("parallel",),
            vmem_limit_bytes=120 * 1024 * 1024,
        ),
    )(A_bf, X)


def _elu(x):
    return jnp.where(x > 0, x, jnp.exp(jnp.minimum(x, 0.0)) - 1.0)


def _post_kernel(zl, zh0, zh1, f0, f1, f2, f3, wd1, bd1, wd2, bd2,
                 wa, ba, qa, zt_ref, zm_ref, scal_ref):
    @pl.when(pl.program_id(0) == 0)
    def _():
        scal_ref[...] = jnp.zeros_like(scal_ref)

    Zl = zl[...]
    Zh0 = zh0[...]
    Zh1 = zh1[...]
    zt_mean_l = 0.5 * (Zl[:, 0:H] + Zl[:, H:2 * H])
    zm_mean_l = 0.5 * (Zl[:, 2 * H:3 * H] + Zl[:, 3 * H:4 * H])
    zt_mean_h = 0.5 * (Zh0[:, 0:H] + Zh0[:, H:2 * H])
    zm_mean_h = 0.5 * (Zh1[:, 0:H] + Zh1[:, H:2 * H])
    z_t = 0.5 * (zt_mean_l + zt_mean_h)
    z_m = 0.5 * (zm_mean_l + zm_mean_h)
    zt_ref[...] = z_t
    zm_ref[...] = z_m

    Wa_ = wa[...]
    ba_ = ba[...]
    qa_ = qa[...]
    s_wt = jnp.sum(
        jnp.tanh(jnp.dot(z_t, Wa_, preferred_element_type=jnp.float32) + ba_)
        * qa_
    )
    s_wm = jnp.sum(
        jnp.tanh(jnp.dot(z_m, Wa_, preferred_element_type=jnp.float32) + ba_)
        * qa_
    )

    Wd1_ = wd1[...]
    bd1_ = bd1[...]
    Wd2_ = wd2[...]
    bd2_ = bd2[...]
    fs = (f0, f1, f2, f3)
    zh_blocks = (Zh0[:, 0:H], Zh0[:, H:2 * H], Zh1[:, 0:H], Zh1[:, H:2 * H])
    sce = [0.0, 0.0]
    for v in range(4):
        zc = jnp.concatenate([Zl[:, v * H:(v + 1) * H], zh_blocks[v]], axis=1)
        hdec = _elu(jnp.dot(zc, Wd1_, preferred_element_type=jnp.float32) + bd1_)
        fea = jnp.dot(hdec, Wd2_, preferred_element_type=jnp.float32) + bd2_
        x = fea
        y = fs[v][...]
        xn = x / (jnp.sqrt(jnp.sum(x * x, axis=1, keepdims=True)) + 1e-8)
        yn = y / (jnp.sqrt(jnp.sum(y * y, axis=1, keepdims=True)) + 1e-8)
        cos = jnp.sum(xn * yn, axis=1)
        sce[v // 2] += jnp.sum((1.0 - cos) ** 2)

    row = jnp.stack([
        jnp.full((128,), s_wt, jnp.float32),
        jnp.full((128,), s_wm, jnp.float32),
        jnp.full((128,), sce[0], jnp.float32),
        jnp.full((128,), sce[1], jnp.float32),
    ])
    scal_ref[...] += row


def _post(Zl, Zh0, Zh1, t0, t1, m0, m1, Wd1, bd1, Wd2, bd2, Wa, ba, qa):
    bm = 2000
    grid = (N // bm,)
    row = lambda i: (i, 0)
    full = lambda i: (0, 0)
    return pl.pallas_call(
        _post_kernel,
        grid=grid,
        in_specs=[
            pl.BlockSpec((bm, 4 * H), row),
            pl.BlockSpec((bm, 2 * H), row),
            pl.BlockSpec((bm, 2 * H), row),
            pl.BlockSpec((bm, F), row),
            pl.BlockSpec((bm, F), row),
            pl.BlockSpec((bm, F), row),
            pl.BlockSpec((bm, F), row),
            pl.BlockSpec((2 * H, H), full),
            pl.BlockSpec((1, H), full),
            pl.BlockSpec((H, F), full),
            pl.BlockSpec((1, F), full),
            pl.BlockSpec((H, H), full),
            pl.BlockSpec((1, H), full),
            pl.BlockSpec((1, H), full),
        ],
        out_specs=[
            pl.BlockSpec((bm, H), row),
            pl.BlockSpec((bm, H), row),
            pl.BlockSpec((4, 128), full),
        ],
        out_shape=[
            jax.ShapeDtypeStruct((N, H), jnp.float32),
            jax.ShapeDtypeStruct((N, H), jnp.float32),
            jax.ShapeDtypeStruct((4, 128), jnp.float32),
        ],
        compiler_params=pltpu.CompilerParams(
            dimension_semantics=("arbitrary",)
        ),
    )(Zl, Zh0, Zh1, t0, t1, m0, m1, Wd1, bd1, Wd2, bd2, Wa, ba, qa)


def _clu_kernel(zt_ref, zm_ref, beta_ref, o_ref):
    b0 = beta_ref[0, 0]
    b1 = beta_ref[0, 1]
    z = b0 * zt_ref[...] + b1 * zm_ref[...]
    mu = z[0:K, :]
    zn = jnp.sum(z * z, axis=1, keepdims=True)
    mn = jnp.sum(mu * mu, axis=1, keepdims=True)
    cross = jnp.dot(z, mu.T, preferred_element_type=jnp.float32)
    d2 = zn - 2.0 * cross + mn.T
    q = 1.0 / (1.0 + d2)
    q = q / jnp.sum(q, axis=1, keepdims=True)
    f = jnp.sum(q, axis=0, keepdims=True)
    p = q * q / f
    p = p / jnp.sum(p, axis=1, keepdims=True)
    kl = jnp.sum(p * jnp.log((p + 1e-8) / (q + 1e-8)), axis=1)
    o_ref[...] = jnp.full((1, 128), jnp.mean(kl), jnp.float32)


def _clustering(z_t, z_m, beta):
    full = lambda: (0, 0)
    return pl.pallas_call(
        _clu_kernel,
        grid=(),
        in_specs=[
            pl.BlockSpec((N, H), full),
            pl.BlockSpec((N, H), full),
            pl.BlockSpec((1, 2), full),
        ],
        out_specs=pl.BlockSpec((1, 128), full),
        out_shape=jax.ShapeDtypeStruct((1, 128), jnp.float32),
    )(z_t, z_m, beta)


def kernel(t0, t1, m0, m1, adj_l, L_h0, L_h1, Wpt0, bpt0, Wpt1, bpt1,
           Wpm0, bpm0, Wpm1, bpm1, Wg0, Wg1, Wa, ba, qa,
           Wl1, bl1, Wl2, bl2, Wh1, bh1, Wh2, bh2, Wc1, bc1, Wc2, bc2,
           Wd1, bd1, Wd2, bd2):
    r = lambda b: b.reshape(1, -1)
    P = _project(t0, t1, m0, m1,
                 (Wpt0, Wpt1, Wpm0, Wpm1),
                 (r(bpt0), r(bpt1), r(bpm0), r(bpm1)), Wg0)

    Q_l, Abf_l = _aggregate_l1(adj_l, P, Wg1)
    Q_h0, Abf_h0 = _aggregate_l1(L_h0, P, Wg1, col_block=2 * H, col_index=0)
    Q_h1, Abf_h1 = _aggregate_l1(L_h1, P, Wg1, col_block=2 * H, col_index=1)
    Z_l = _aggregate_l2(Abf_l, Q_l)
    Z_h0 = _aggregate_l2(Abf_h0, Q_h0)
    Z_h1 = _aggregate_l2(Abf_h1, Q_h1)

    z_t, z_m, scal = _post(Z_l, Z_h0, Z_h1, t0, t1, m0, m1,
                           Wd1, r(bd1), Wd2, r(bd2), Wa, r(ba), r(qa))

    w_t = scal[0, 0] / N
    w_m = scal[1, 0] / N
    beta = jax.nn.softmax(jnp.stack([w_t, w_m]))
    loss_rec = (scal[2, 0] + scal[3, 0]) / (2.0 * N) / 4.0

    loss_clu = _clustering(z_t, z_m, beta.reshape(1, 2))[0, 0]

    return loss_rec + loss_clu


# speedup vs baseline: 1.0522x; 1.0522x over previous
import functools

import jax
import jax.numpy as jnp
from jax.experimental import pallas as pl
from jax.experimental.pallas import tpu as pltpu

N = 10000
F = 128
H = 64
K = 16


def _proj_kernel(x0, x1, x2, x3, w0, w1, w2, w3, b0, b1, b2, b3, wg0, p_ref):
    xs = (x0, x1, x2, x3)
    ws = (w0, w1, w2, w3)
    bs = (b0, b1, b2, b3)
    g = wg0[...]
    for v in range(4):
        z = jnp.maximum(
            jnp.dot(xs[v][...], ws[v][...], preferred_element_type=jnp.float32)
            + bs[v][...],
            0.0,
        )
        p_ref[:, v * H:(v + 1) * H] = jnp.dot(
            z, g, preferred_element_type=jnp.float32
        ).astype(jnp.bfloat16)


def _project(t0, t1, m0, m1, Wps, bps, Wg0):
    bm = 2000
    grid = (N // bm,)
    row = lambda i: (i, 0)
    full = lambda i: (0, 0)
    return pl.pallas_call(
        _proj_kernel,
        grid=grid,
        in_specs=[pl.BlockSpec((bm, F), row)] * 4
        + [pl.BlockSpec((F, H), full)] * 4
        + [pl.BlockSpec((1, H), full)] * 4
        + [pl.BlockSpec((H, H), full)],
        out_specs=pl.BlockSpec((bm, 4 * H), row),
        out_shape=jax.ShapeDtypeStruct((N, 4 * H), jnp.bfloat16),
        compiler_params=pltpu.CompilerParams(
            dimension_semantics=("parallel",)
        ),
    )(t0, t1, m0, m1, *Wps, *bps, Wg0)


def _agg_kernel_epi(a_ref, x_ref, wg1_ref, o_ref, *, cb):
    acc = jnp.dot(
        a_ref[...].astype(jnp.bfloat16),
        x_ref[...],
        preferred_element_type=jnp.float32,
    )
    g = wg1_ref[...]
    for j in range(cb // H):
        o_ref[:, j * H:(j + 1) * H] = jnp.dot(
            jnp.maximum(acc[:, j * H:(j + 1) * H], 0.0),
            g,
            preferred_element_type=jnp.float32,
        ).astype(jnp.bfloat16)


def _agg_kernel_plain(a_ref, x_ref, o_ref):
    o_ref[...] = jnp.dot(
        a_ref[...].astype(jnp.bfloat16),
        x_ref[...],
        preferred_element_type=jnp.float32,
    ).astype(jnp.bfloat16)


def _aggregate(A, X, Wg1=None, col_block=None, col_index=0, bm=400):
    cb = col_block if col_block is not None else X.shape[1]
    a_spec = pl.BlockSpec((bm, N), lambda i: (i, 0))
    x_spec = pl.BlockSpec((N, cb), lambda i: (0, col_index))
    o_spec = pl.BlockSpec((bm, cb), lambda i: (i, 0))
    if Wg1 is None:
        body = _agg_kernel_plain
        in_specs = [a_spec, x_spec]
        ops = (A, X)
        out_dtype = jnp.bfloat16
    else:
        body = functools.partial(_agg_kernel_epi, cb=cb)
        in_specs = [a_spec, x_spec, pl.BlockSpec((H, H), lambda i: (0, 0))]
        ops = (A, X, Wg1)
        out_dtype = jnp.bfloat16
    return pl.pallas_call(
        body,
        grid=(N // bm,),
        in_specs=in_specs,
        out_specs=o_spec,
        out_shape=jax.ShapeDtypeStruct((N, cb), out_dtype),
        compiler_params=pltpu.CompilerParams(
            dimension_semantics=("parallel",)
        ),
    )(*ops)


def _elu(x):
    return jnp.where(x > 0, x, jnp.exp(jnp.minimum(x, 0.0)) - 1.0)


def _post_kernel(zl, zh0, zh1, f0, f1, f2, f3, wd1, bd1, wd2, bd2,
                 wa, ba, qa, zt_ref, zm_ref, scal_ref):
    @pl.when(pl.program_id(0) == 0)
    def _():
        scal_ref[...] = jnp.zeros_like(scal_ref)

    Zl = zl[...].astype(jnp.float32)
    Zh0 = zh0[...].astype(jnp.float32)
    Zh1 = zh1[...].astype(jnp.float32)
    zt_mean_l = 0.5 * (Zl[:, 0:H] + Zl[:, H:2 * H])
    zm_mean_l = 0.5 * (Zl[:, 2 * H:3 * H] + Zl[:, 3 * H:4 * H])
    zt_mean_h = 0.5 * (Zh0[:, 0:H] + Zh0[:, H:2 * H])
    zm_mean_h = 0.5 * (Zh1[:, 0:H] + Zh1[:, H:2 * H])
    z_t = 0.5 * (zt_mean_l + zt_mean_h)
    z_m = 0.5 * (zm_mean_l + zm_mean_h)
    zt_ref[...] = z_t
    zm_ref[...] = z_m

    Wa_ = wa[...]
    ba_ = ba[...]
    qa_ = qa[...]
    s_wt = jnp.sum(
        jnp.tanh(jnp.dot(z_t, Wa_, preferred_element_type=jnp.float32) + ba_)
        * qa_
    )
    s_wm = jnp.sum(
        jnp.tanh(jnp.dot(z_m, Wa_, preferred_element_type=jnp.float32) + ba_)
        * qa_
    )

    Wd1_ = wd1[...]
    bd1_ = bd1[...]
    Wd2_ = wd2[...]
    bd2_ = bd2[...]
    fs = (f0, f1, f2, f3)
    zh_blocks = (Zh0[:, 0:H], Zh0[:, H:2 * H], Zh1[:, 0:H], Zh1[:, H:2 * H])
    sce = [0.0, 0.0]
    for v in range(4):
        zc = jnp.concatenate([Zl[:, v * H:(v + 1) * H], zh_blocks[v]], axis=1)
        hdec = _elu(jnp.dot(zc, Wd1_, preferred_element_type=jnp.float32) + bd1_)
        fea = jnp.dot(hdec, Wd2_, preferred_element_type=jnp.float32) + bd2_
        x = fea
        y = fs[v][...]
        xn = x / (jnp.sqrt(jnp.sum(x * x, axis=1, keepdims=True)) + 1e-8)
        yn = y / (jnp.sqrt(jnp.sum(y * y, axis=1, keepdims=True)) + 1e-8)
        cos = jnp.sum(xn * yn, axis=1)
        sce[v // 2] += jnp.sum((1.0 - cos) ** 2)

    row = jnp.stack([
        jnp.full((128,), s_wt, jnp.float32),
        jnp.full((128,), s_wm, jnp.float32),
        jnp.full((128,), sce[0], jnp.float32),
        jnp.full((128,), sce[1], jnp.float32),
    ])
    scal_ref[...] += row


def _post(Zl, Zh0, Zh1, t0, t1, m0, m1, Wd1, bd1, Wd2, bd2, Wa, ba, qa):
    bm = 2000
    grid = (N // bm,)
    row = lambda i: (i, 0)
    full = lambda i: (0, 0)
    return pl.pallas_call(
        _post_kernel,
        grid=grid,
        in_specs=[
            pl.BlockSpec((bm, 4 * H), row),
            pl.BlockSpec((bm, 2 * H), row),
            pl.BlockSpec((bm, 2 * H), row),
            pl.BlockSpec((bm, F), row),
            pl.BlockSpec((bm, F), row),
            pl.BlockSpec((bm, F), row),
            pl.BlockSpec((bm, F), row),
            pl.BlockSpec((2 * H, H), full),
            pl.BlockSpec((1, H), full),
            pl.BlockSpec((H, F), full),
            pl.BlockSpec((1, F), full),
            pl.BlockSpec((H, H), full),
            pl.BlockSpec((1, H), full),
            pl.BlockSpec((1, H), full),
        ],
        out_specs=[
            pl.BlockSpec((bm, H), row),
            pl.BlockSpec((bm, H), row),
            pl.BlockSpec((4, 128), full),
        ],
        out_shape=[
            jax.ShapeDtypeStruct((N, H), jnp.float32),
            jax.ShapeDtypeStruct((N, H), jnp.float32),
            jax.ShapeDtypeStruct((4, 128), jnp.float32),
        ],
        compiler_params=pltpu.CompilerParams(
            dimension_semantics=("arbitrary",)
        ),
    )(Zl, Zh0, Zh1, t0, t1, m0, m1, Wd1, bd1, Wd2, bd2, Wa, ba, qa)


def _clu_kernel(zt_ref, zm_ref, scal_ref, o_ref):
    s_wt = scal_ref[0, 0] / N
    s_wm = scal_ref[1, 0] / N
    m = jnp.maximum(s_wt, s_wm)
    e0 = jnp.exp(s_wt - m)
    e1 = jnp.exp(s_wm - m)
    b0 = e0 / (e0 + e1)
    b1 = e1 / (e0 + e1)
    loss_rec = (scal_ref[2, 0] + scal_ref[3, 0]) / (2.0 * N) / 4.0
    z = b0 * zt_ref[...] + b1 * zm_ref[...]
    mu = z[0:K, :]
    zn = jnp.sum(z * z, axis=1, keepdims=True)
    mn = jnp.sum(mu * mu, axis=1, keepdims=True)
    cross = jnp.dot(z, mu.T, preferred_element_type=jnp.float32)
    d2 = zn - 2.0 * cross + mn.T
    q = 1.0 / (1.0 + d2)
    q = q / jnp.sum(q, axis=1, keepdims=True)
    f = jnp.sum(q, axis=0, keepdims=True)
    p = q * q / f
    p = p / jnp.sum(p, axis=1, keepdims=True)
    kl = jnp.sum(p * jnp.log((p + 1e-8) / (q + 1e-8)), axis=1)
    o_ref[...] = jnp.full((1, 128), loss_rec + jnp.mean(kl), jnp.float32)


def _clustering(z_t, z_m, scal):
    full = lambda: (0, 0)
    return pl.pallas_call(
        _clu_kernel,
        grid=(),
        in_specs=[
            pl.BlockSpec((N, H), full),
            pl.BlockSpec((N, H), full),
            pl.BlockSpec((4, 128), full),
        ],
        out_specs=pl.BlockSpec((1, 128), full),
        out_shape=jax.ShapeDtypeStruct((1, 128), jnp.float32),
    )(z_t, z_m, scal)


def kernel(t0, t1, m0, m1, adj_l, L_h0, L_h1, Wpt0, bpt0, Wpt1, bpt1,
           Wpm0, bpm0, Wpm1, bpm1, Wg0, Wg1, Wa, ba, qa,
           Wl1, bl1, Wl2, bl2, Wh1, bh1, Wh2, bh2, Wc1, bc1, Wc2, bc2,
           Wd1, bd1, Wd2, bd2):
    r = lambda b: b.reshape(1, -1)
    P = _project(t0, t1, m0, m1,
                 (Wpt0, Wpt1, Wpm0, Wpm1),
                 (r(bpt0), r(bpt1), r(bpm0), r(bpm1)), Wg0)

    Q_l = _aggregate(adj_l, P, Wg1=Wg1)
    Q_h0 = _aggregate(L_h0, P, Wg1=Wg1, col_block=2 * H, col_index=0)
    Q_h1 = _aggregate(L_h1, P, Wg1=Wg1, col_block=2 * H, col_index=1)
    Z_l = _aggregate(adj_l, Q_l)
    Z_h0 = _aggregate(L_h0, Q_h0)
    Z_h1 = _aggregate(L_h1, Q_h1)

    z_t, z_m, scal = _post(Z_l, Z_h0, Z_h1, t0, t1, m0, m1,
                           Wd1, r(bd1), Wd2, r(bd2), Wa, r(ba), r(qa))

    return _clustering(z_t, z_m, scal)[0, 0]


# speedup vs baseline: 1.0740x; 1.0207x over previous
import functools

import jax
import jax.numpy as jnp
from jax.experimental import pallas as pl
from jax.experimental.pallas import tpu as pltpu

N = 10000
F = 128
H = 64
K = 16


def _proj_kernel(x0, x1, x2, x3, w0, w1, w2, w3, b0, b1, b2, b3, wg0, p_ref):
    xs = (x0, x1, x2, x3)
    ws = (w0, w1, w2, w3)
    bs = (b0, b1, b2, b3)
    g = wg0[...]
    for v in range(4):
        z = jnp.maximum(
            jnp.dot(xs[v][...], ws[v][...], preferred_element_type=jnp.float32)
            + bs[v][...],
            0.0,
        )
        p_ref[:, v * H:(v + 1) * H] = jnp.dot(
            z, g, preferred_element_type=jnp.float32
        ).astype(jnp.bfloat16)


def _project(t0, t1, m0, m1, Wps, bps, Wg0):
    bm = 2000
    grid = (N // bm,)
    row = lambda i: (i, 0)
    full = lambda i: (0, 0)
    return pl.pallas_call(
        _proj_kernel,
        grid=grid,
        in_specs=[pl.BlockSpec((bm, F), row)] * 4
        + [pl.BlockSpec((F, H), full)] * 4
        + [pl.BlockSpec((1, H), full)] * 4
        + [pl.BlockSpec((H, H), full)],
        out_specs=pl.BlockSpec((bm, 4 * H), row),
        out_shape=jax.ShapeDtypeStruct((N, 4 * H), jnp.bfloat16),
        compiler_params=pltpu.CompilerParams(
            dimension_semantics=("parallel",)
        ),
    )(t0, t1, m0, m1, *Wps, *bps, Wg0)


def _agg_kernel_epi(a_ref, x_ref, wg1_ref, o_ref, *, cb):
    acc = jnp.dot(
        a_ref[...].astype(jnp.bfloat16),
        x_ref[...],
        preferred_element_type=jnp.float32,
    )
    g = wg1_ref[...]
    for j in range(cb // H):
        o_ref[:, j * H:(j + 1) * H] = jnp.dot(
            jnp.maximum(acc[:, j * H:(j + 1) * H], 0.0),
            g,
            preferred_element_type=jnp.float32,
        ).astype(jnp.bfloat16)


def _agg_kernel_plain(a_ref, x_ref, o_ref):
    o_ref[...] = jnp.dot(
        a_ref[...].astype(jnp.bfloat16),
        x_ref[...],
        preferred_element_type=jnp.float32,
    ).astype(jnp.bfloat16)


def _aggregate(A, X, Wg1=None, col_block=None, col_index=0, bm=400):
    cb = col_block if col_block is not None else X.shape[1]
    a_spec = pl.BlockSpec((bm, N), lambda i: (i, 0))
    x_spec = pl.BlockSpec((N, cb), lambda i: (0, col_index))
    o_spec = pl.BlockSpec((bm, cb), lambda i: (i, 0))
    if Wg1 is None:
        body = _agg_kernel_plain
        in_specs = [a_spec, x_spec]
        ops = (A, X)
        out_dtype = jnp.bfloat16
    else:
        body = functools.partial(_agg_kernel_epi, cb=cb)
        in_specs = [a_spec, x_spec, pl.BlockSpec((H, H), lambda i: (0, 0))]
        ops = (A, X, Wg1)
        out_dtype = jnp.bfloat16
    return pl.pallas_call(
        body,
        grid=(N // bm,),
        in_specs=in_specs,
        out_specs=o_spec,
        out_shape=jax.ShapeDtypeStruct((N, cb), out_dtype),
        compiler_params=pltpu.CompilerParams(
            dimension_semantics=("parallel",)
        ),
    )(*ops)


def _elu(x):
    return jnp.where(x > 0, x, jnp.exp(jnp.minimum(x, 0.0)) - 1.0)


def _l2post_kernel(a_ref, qh1_ref, zl, zh0, f0, f1, f2, f3,
                   wd1, bd1, wd2, bd2, wa, ba, qa,
                   zt_ref, zm_ref, scal_ref):
    @pl.when(pl.program_id(0) == 0)
    def _():
        scal_ref[...] = jnp.zeros_like(scal_ref)

    Zh1 = jnp.dot(
        a_ref[...].astype(jnp.bfloat16),
        qh1_ref[...],
        preferred_element_type=jnp.float32,
    )
    Zl = zl[...].astype(jnp.float32)
    Zh0 = zh0[...].astype(jnp.float32)
    zt_mean_l = 0.5 * (Zl[:, 0:H] + Zl[:, H:2 * H])
    zm_mean_l = 0.5 * (Zl[:, 2 * H:3 * H] + Zl[:, 3 * H:4 * H])
    zt_mean_h = 0.5 * (Zh0[:, 0:H] + Zh0[:, H:2 * H])
    zm_mean_h = 0.5 * (Zh1[:, 0:H] + Zh1[:, H:2 * H])
    z_t = 0.5 * (zt_mean_l + zt_mean_h)
    z_m = 0.5 * (zm_mean_l + zm_mean_h)
    zt_ref[...] = z_t
    zm_ref[...] = z_m

    Wa_ = wa[...]
    ba_ = ba[...]
    qa_ = qa[...]
    s_wt = jnp.sum(
        jnp.tanh(jnp.dot(z_t, Wa_, preferred_element_type=jnp.float32) + ba_)
        * qa_
    )
    s_wm = jnp.sum(
        jnp.tanh(jnp.dot(z_m, Wa_, preferred_element_type=jnp.float32) + ba_)
        * qa_
    )

    Wd1_ = wd1[...]
    bd1_ = bd1[...]
    Wd2_ = wd2[...]
    bd2_ = bd2[...]
    fs = (f0, f1, f2, f3)
    zh_blocks = (Zh0[:, 0:H], Zh0[:, H:2 * H], Zh1[:, 0:H], Zh1[:, H:2 * H])
    sce = [0.0, 0.0]
    for v in range(4):
        zc = jnp.concatenate([Zl[:, v * H:(v + 1) * H], zh_blocks[v]], axis=1)
        hdec = _elu(jnp.dot(zc, Wd1_, preferred_element_type=jnp.float32) + bd1_)
        fea = jnp.dot(hdec, Wd2_, preferred_element_type=jnp.float32) + bd2_
        x = fea
        y = fs[v][...]
        xn = x / (jnp.sqrt(jnp.sum(x * x, axis=1, keepdims=True)) + 1e-8)
        yn = y / (jnp.sqrt(jnp.sum(y * y, axis=1, keepdims=True)) + 1e-8)
        cos = jnp.sum(xn * yn, axis=1)
        sce[v // 2] += jnp.sum((1.0 - cos) ** 2)

    row = jnp.stack([
        jnp.full((128,), s_wt, jnp.float32),
        jnp.full((128,), s_wm, jnp.float32),
        jnp.full((128,), sce[0], jnp.float32),
        jnp.full((128,), sce[1], jnp.float32),
    ])
    scal_ref[...] += row


def _l2post(A, Q_h1, Zl, Zh0, t0, t1, m0, m1,
            Wd1, bd1, Wd2, bd2, Wa, ba, qa):
    bm = 400
    grid = (N // bm,)
    row = lambda i: (i, 0)
    full = lambda i: (0, 0)
    return pl.pallas_call(
        _l2post_kernel,
        grid=grid,
        in_specs=[
            pl.BlockSpec((bm, N), row),
            pl.BlockSpec((N, 2 * H), full),
            pl.BlockSpec((bm, 4 * H), row),
            pl.BlockSpec((bm, 2 * H), row),
            pl.BlockSpec((bm, F), row),
            pl.BlockSpec((bm, F), row),
            pl.BlockSpec((bm, F), row),
            pl.BlockSpec((bm, F), row),
            pl.BlockSpec((2 * H, H), full),
            pl.BlockSpec((1, H), full),
            pl.BlockSpec((H, F), full),
            pl.BlockSpec((1, F), full),
            pl.BlockSpec((H, H), full),
            pl.BlockSpec((1, H), full),
            pl.BlockSpec((1, H), full),
        ],
        out_specs=[
            pl.BlockSpec((bm, H), row),
            pl.BlockSpec((bm, H), row),
            pl.BlockSpec((4, 128), full),
        ],
        out_shape=[
            jax.ShapeDtypeStruct((N, H), jnp.float32),
            jax.ShapeDtypeStruct((N, H), jnp.float32),
            jax.ShapeDtypeStruct((4, 128), jnp.float32),
        ],
        compiler_params=pltpu.CompilerParams(
            dimension_semantics=("arbitrary",)
        ),
    )(A, Q_h1, Zl, Zh0, t0, t1, m0, m1,
      Wd1, bd1, Wd2, bd2, Wa, ba, qa)


def _clu_kernel(zt_ref, zm_ref, scal_ref, o_ref):
    s_wt = scal_ref[0, 0] / N
    s_wm = scal_ref[1, 0] / N
    m = jnp.maximum(s_wt, s_wm)
    e0 = jnp.exp(s_wt - m)
    e1 = jnp.exp(s_wm - m)
    b0 = e0 / (e0 + e1)
    b1 = e1 / (e0 + e1)
    loss_rec = (scal_ref[2, 0] + scal_ref[3, 0]) / (2.0 * N) / 4.0
    z = b0 * zt_ref[...] + b1 * zm_ref[...]
    mu = z[0:K, :]
    zn = jnp.sum(z * z, axis=1, keepdims=True)
    mn = jnp.sum(mu * mu, axis=1, keepdims=True)
    cross = jnp.dot(z, mu.T, preferred_element_type=jnp.float32)
    d2 = zn - 2.0 * cross + mn.T
    q = 1.0 / (1.0 + d2)
    q = q / jnp.sum(q, axis=1, keepdims=True)
    f = jnp.sum(q, axis=0, keepdims=True)
    p = q * q / f
    p = p / jnp.sum(p, axis=1, keepdims=True)
    kl = jnp.sum(p * jnp.log((p + 1e-8) / (q + 1e-8)), axis=1)
    o_ref[...] = jnp.full((1, 128), loss_rec + jnp.mean(kl), jnp.float32)


def _clustering(z_t, z_m, scal):
    full = lambda: (0, 0)
    return pl.pallas_call(
        _clu_kernel,
        grid=(),
        in_specs=[
            pl.BlockSpec((N, H), full),
            pl.BlockSpec((N, H), full),
            pl.BlockSpec((4, 128), full),
        ],
        out_specs=pl.BlockSpec((1, 128), full),
        out_shape=jax.ShapeDtypeStruct((1, 128), jnp.float32),
    )(z_t, z_m, scal)


def kernel(t0, t1, m0, m1, adj_l, L_h0, L_h1, Wpt0, bpt0, Wpt1, bpt1,
           Wpm0, bpm0, Wpm1, bpm1, Wg0, Wg1, Wa, ba, qa,
           Wl1, bl1, Wl2, bl2, Wh1, bh1, Wh2, bh2, Wc1, bc1, Wc2, bc2,
           Wd1, bd1, Wd2, bd2):
    r = lambda b: b.reshape(1, -1)
    P = _project(t0, t1, m0, m1,
                 (Wpt0, Wpt1, Wpm0, Wpm1),
                 (r(bpt0), r(bpt1), r(bpm0), r(bpm1)), Wg0)

    Q_l = _aggregate(adj_l, P, Wg1=Wg1)
    Q_h0 = _aggregate(L_h0, P, Wg1=Wg1, col_block=2 * H, col_index=0)
    Q_h1 = _aggregate(L_h1, P, Wg1=Wg1, col_block=2 * H, col_index=1)
    Z_l = _aggregate(adj_l, Q_l)
    Z_h0 = _aggregate(L_h0, Q_h0)

    z_t, z_m, scal = _l2post(L_h1, Q_h1, Z_l, Z_h0, t0, t1, m0, m1,
                             Wd1, r(bd1), Wd2, r(bd2), Wa, r(ba), r(qa))

    return _clustering(z_t, z_m, scal)[0, 0]
